# baseline (speedup 1.0000x reference)
import functools

import jax
import jax.numpy as jnp
from jax.experimental import pallas as pl
from jax.experimental.pallas import tpu as pltpu


_VMEM_LIMIT = 64 * 1024 * 1024


def _prep_kernel(*refs):
    k = len(refs) // 2
    in_refs, out_refs = refs[:k], refs[k:]
    for i_ref, o_ref in zip(in_refs[:-1], out_refs[:-1]):
        o_ref[...] = i_ref[...].astype(jnp.bfloat16)
    out_refs[-1][...] = jnp.transpose(in_refs[-1][...])


def _prep(weights, cluster_layer):
    ins = list(weights) + [cluster_layer]
    out_shape = tuple(jax.ShapeDtypeStruct(w.shape, jnp.bfloat16)
                      for w in weights)
    out_shape += (jax.ShapeDtypeStruct(
        (cluster_layer.shape[1], cluster_layer.shape[0]), jnp.float32),)
    return pl.pallas_call(
        _prep_kernel,
        out_shape=out_shape,
        compiler_params=pltpu.CompilerParams(
            vmem_limit_bytes=_VMEM_LIMIT,
        ),
    )(*ins)


def _row_tile(n, pref=1024):
    for t in (pref, 512, 256, 128, 64, 32, 16, 8):
        if t <= n and n % t == 0:
            return t
    return n


def _ae_kernel(x_ref,
               ew1_ref, eb1_ref, ew2_ref, eb2_ref, ew3_ref, eb3_ref,
               zw_ref, zb_ref,
               dw1_ref, db1_ref, dw2_ref, db2_ref, dw3_ref, db3_ref,
               xw_ref, xb_ref,
               gw1_ref, gw4_ref, gw5_ref, ct_ref,
               xbar_ref, z_ref, h1_ref, h2_ref, t4_ref, t5_ref,
               sup1_ref, q_ref,
               *, sigma, v):
    x = x_ref[...].astype(jnp.bfloat16)

    def lin(a_bf, w_ref, b_ref, relu):
        y = jnp.dot(a_bf, w_ref[...], preferred_element_type=jnp.float32)
        y = y + b_ref[...]
        return jnp.maximum(y, 0.0) if relu else y

    h1 = lin(x, ew1_ref, eb1_ref, True)
    h1b = h1.astype(jnp.bfloat16)
    h2 = lin(h1b, ew2_ref, eb2_ref, True)
    h2b = h2.astype(jnp.bfloat16)
    h3 = lin(h2b, ew3_ref, eb3_ref, True)
    h3b = h3.astype(jnp.bfloat16)
    z = lin(h3b, zw_ref, zb_ref, False)
    zb = z.astype(jnp.bfloat16)
    d1 = lin(zb, dw1_ref, db1_ref, True).astype(jnp.bfloat16)
    d2 = lin(d1, dw2_ref, db2_ref, True).astype(jnp.bfloat16)
    d3 = lin(d2, dw3_ref, db3_ref, True).astype(jnp.bfloat16)
    x_bar = lin(d3, xw_ref, xb_ref, False)

    xbar_ref[...] = x_bar
    z_ref[...] = z
    h1_ref[...] = h1b
    h2_ref[...] = h2b
    sup1_ref[...] = jnp.dot(x, gw1_ref[...],
                            preferred_element_type=jnp.float32).astype(jnp.bfloat16)
    t4_ref[...] = (sigma * jnp.dot(h3b, gw4_ref[...],
                                   preferred_element_type=jnp.float32)
                   ).astype(jnp.bfloat16)
    t5_ref[...] = (sigma * jnp.dot(zb, gw5_ref[...],
                                   preferred_element_type=jnp.float32)
                   ).astype(jnp.bfloat16)

    ct = ct_ref[...]
    zz = jnp.sum(z * z, axis=1, keepdims=True)
    cc = jnp.sum(ct * ct, axis=0, keepdims=True)
    d = zz + cc - 2.0 * jnp.dot(z, ct, preferred_element_type=jnp.float32)
    d = jnp.maximum(d, 0.0)
    q = 1.0 / (1.0 + d / v)
    expo = (v + 1.0) / 2.0
    if expo != 1.0:
        q = q ** expo
    q_ref[...] = q / jnp.sum(q, axis=1, keepdims=True)


def _ae_forward(x, p_bf, biases, gw1_bf, gw4_bf, gw5_bf, c_t, *, sigma, v):
    n, n_input = x.shape
    tm = _row_tile(n, pref=512)
    grid = (n // tm,)

    w_names = ("enc_1", "enc_2", "enc_3", "z", "dec_1", "dec_2", "dec_3", "x_bar")
    args = [x]
    in_specs = [pl.BlockSpec((tm, n_input), lambda i: (i, 0))]
    for name in w_names:
        w = p_bf[name]
        b = biases[name]
        args += [w, b]
        in_specs += [pl.BlockSpec(w.shape, lambda i: (0, 0)),
                     pl.BlockSpec(b.shape, lambda i: (0, 0))]
    args += [gw1_bf, gw4_bf, gw5_bf, c_t]
    in_specs += [pl.BlockSpec(gw1_bf.shape, lambda i: (0, 0)),
                 pl.BlockSpec(gw4_bf.shape, lambda i: (0, 0)),
                 pl.BlockSpec(gw5_bf.shape, lambda i: (0, 0)),
                 pl.BlockSpec(c_t.shape, lambda i: (0, 0))]

    d_h1 = p_bf["enc_1"].shape[1]
    d_h2 = p_bf["enc_2"].shape[1]
    d_z = p_bf["z"].shape[1]
    d_t4 = gw4_bf.shape[1]
    d_t5 = gw5_bf.shape[1]
    d_s1 = gw1_bf.shape[1]
    n_k = c_t.shape[1]

    out_shape = (jax.ShapeDtypeStruct((n, n_input), jnp.float32),
                 jax.ShapeDtypeStruct((n, d_z), jnp.float32),
                 jax.ShapeDtypeStruct((n, d_h1), jnp.bfloat16),
                 jax.ShapeDtypeStruct((n, d_h2), jnp.bfloat16),
                 jax.ShapeDtypeStruct((n, d_t4), jnp.bfloat16),
                 jax.ShapeDtypeStruct((n, d_t5), jnp.bfloat16),
                 jax.ShapeDtypeStruct((n, d_s1), jnp.bfloat16),
                 jax.ShapeDtypeStruct((n, n_k), jnp.float32))
    out_specs = tuple(pl.BlockSpec((tm, s.shape[1]), lambda i: (i, 0))
                      for s in out_shape)

    return pl.pallas_call(
        functools.partial(_ae_kernel, sigma=sigma, v=v),
        out_shape=out_shape,
        grid_spec=pltpu.PrefetchScalarGridSpec(
            num_scalar_prefetch=0,
            grid=grid,
            in_specs=in_specs,
            out_specs=out_specs,
        ),
        compiler_params=pltpu.CompilerParams(
            dimension_semantics=("parallel",),
            vmem_limit_bytes=_VMEM_LIMIT,
        ),
    )(*args)


def _gnn_first_kernel(adj_ref, sup_ref, tra_ref, w_ref,
                      out_ref, a01_ref, c_ref, *, sigma):
    a = adj_ref[...]
    c = jnp.max(a, axis=1, keepdims=True)
    acc = jnp.dot(a.astype(jnp.bfloat16), sup_ref[...],
                  preferred_element_type=jnp.float32)
    h = jnp.maximum(acc, 0.0)
    feat = (1.0 - sigma) * h + sigma * tra_ref[...].astype(jnp.float32)
    out_ref[...] = jnp.dot(feat.astype(jnp.bfloat16), w_ref[...],
                           preferred_element_type=jnp.float32).astype(jnp.bfloat16)
    a01_ref[...] = (a > 0.0).astype(jnp.int8)
    c_ref[...] = jnp.broadcast_to(c, (c.shape[0], 8))


def _gnn_first(adj, support, tra, w_next_bf, *, sigma):
    n = adj.shape[0]
    f_sup = support.shape[1]
    f_tra = tra.shape[1]
    f_next = w_next_bf.shape[1]
    tm = _row_tile(n)
    grid = (n // tm,)
    return pl.pallas_call(
        functools.partial(_gnn_first_kernel, sigma=sigma),
        out_shape=(jax.ShapeDtypeStruct((n, f_next), jnp.bfloat16),
                   jax.ShapeDtypeStruct((n, n), jnp.int8),
                   jax.ShapeDtypeStruct((n, 8), jnp.float32)),
        grid_spec=pltpu.PrefetchScalarGridSpec(
            num_scalar_prefetch=0,
            grid=grid,
            in_specs=[
                pl.BlockSpec((tm, n), lambda i: (i, 0)),
                pl.BlockSpec((n, f_sup), lambda i: (0, 0)),
                pl.BlockSpec((tm, f_tra), lambda i: (i, 0)),
                pl.BlockSpec((f_sup, f_next), lambda i: (0, 0)),
            ],
            out_specs=(pl.BlockSpec((tm, f_next), lambda i: (i, 0)),
                       pl.BlockSpec((tm, n), lambda i: (i, 0)),
                       pl.BlockSpec((tm, 8), lambda i: (i, 0))),
        ),
        compiler_params=pltpu.CompilerParams(
            dimension_semantics=("parallel",),
            vmem_limit_bytes=_VMEM_LIMIT,
        ),
    )(adj, support, tra, w_next_bf)


def _gnn_kernel(a01_ref, c_ref, sup_ref, tra_ref, *args, sigma, mode):
    out_ref = args[-1]
    a01 = a01_ref[...].astype(jnp.bfloat16)
    c = c_ref[:, 0:1]
    acc = jnp.dot(a01, sup_ref[...],
                  preferred_element_type=jnp.float32) * c
    if mode == "feat":
        h = jnp.maximum(acc, 0.0)
        out = (1.0 - sigma) * h + sigma * tra_ref[...].astype(jnp.float32)
    elif mode == "expand":
        win_ref, wout_ref = args[0], args[1]
        h = jnp.maximum(jnp.dot(acc.astype(jnp.bfloat16), win_ref[...],
                                preferred_element_type=jnp.float32), 0.0)
        hw = jnp.dot(h.astype(jnp.bfloat16), wout_ref[...],
                     preferred_element_type=jnp.float32)
        out = (1.0 - sigma) * hw + tra_ref[...].astype(jnp.float32)
    else:
        w_ref = args[0]
        h = jnp.maximum(acc, 0.0)
        hw = jnp.dot(h.astype(jnp.bfloat16), w_ref[...],
                     preferred_element_type=jnp.float32)
        out = (1.0 - sigma) * hw + tra_ref[...].astype(jnp.float32)
    out_ref[...] = out.astype(jnp.bfloat16)


def _gnn_chain(a01, c, support, tra, weights, f_next, *, sigma, mode):
    n = a01.shape[0]
    f_sup = support.shape[1]
    f_tra = tra.shape[1]
    tm = _row_tile(n)
    grid = (n // tm,)
    in_specs = [
        pl.BlockSpec((tm, n), lambda i: (i, 0)),
        pl.BlockSpec((tm, 8), lambda i: (i, 0)),
        pl.BlockSpec((n, f_sup), lambda i: (0, 0)),
        pl.BlockSpec((tm, f_tra), lambda i: (i, 0)),
    ]
    for w in weights:
        in_specs.append(pl.BlockSpec(w.shape, lambda i: (0, 0)))
    return pl.pallas_call(
        functools.partial(_gnn_kernel, sigma=sigma, mode=mode),
        out_shape=jax.ShapeDtypeStruct((n, f_next), jnp.bfloat16),
        grid_spec=pltpu.PrefetchScalarGridSpec(
            num_scalar_prefetch=0,
            grid=grid,
            in_specs=in_specs,
            out_specs=pl.BlockSpec((tm, f_next), lambda i: (i, 0)),
        ),
        compiler_params=pltpu.CompilerParams(
            dimension_semantics=("parallel",),
            vmem_limit_bytes=_VMEM_LIMIT,
        ),
    )(a01, c, support, tra, *weights)


def _final_kernel(a01_ref, c_ref, sup_ref, pred_ref):
    a01 = a01_ref[...].astype(jnp.bfloat16)
    c = c_ref[:, 0:1]
    h = jnp.dot(a01, sup_ref[...],
                preferred_element_type=jnp.float32) * c
    m = jnp.max(h, axis=1, keepdims=True)
    e = jnp.exp(h - m)
    pred_ref[...] = e / jnp.sum(e, axis=1, keepdims=True)


def _gnn_final(a01, c, support):
    n = a01.shape[0]
    f_sup = support.shape[1]
    tm = _row_tile(n)
    grid = (n // tm,)
    return pl.pallas_call(
        _final_kernel,
        out_shape=jax.ShapeDtypeStruct((n, f_sup), jnp.float32),
        grid_spec=pltpu.PrefetchScalarGridSpec(
            num_scalar_prefetch=0,
            grid=grid,
            in_specs=[
                pl.BlockSpec((tm, n), lambda i: (i, 0)),
                pl.BlockSpec((tm, 8), lambda i: (i, 0)),
                pl.BlockSpec((n, f_sup), lambda i: (0, 0)),
            ],
            out_specs=pl.BlockSpec((tm, f_sup), lambda i: (i, 0)),
        ),
        compiler_params=pltpu.CompilerParams(
            dimension_semantics=("parallel",),
            vmem_limit_bytes=_VMEM_LIMIT,
        ),
    )(a01, c, support)


def kernel(x, adj, enc_1_w, enc_1_b, enc_2_w, enc_2_b, enc_3_w, enc_3_b,
           z_w, z_b, dec_1_w, dec_1_b, dec_2_w, dec_2_b, dec_3_w, dec_3_b,
           x_bar_w, x_bar_b,
           gnn_1_w, gnn_2_w, gnn_3_w, gnn_4_w, gnn_5_w,
           cluster_layer):
    sigma, v = 0.5, 1.0

    (ew1, ew2, ew3, zw, dw1, dw2, dw3, xw,
     gw1, gw2, gw3, gw4, gw5, c_t) = _prep(
        (enc_1_w, enc_2_w, enc_3_w, z_w, dec_1_w, dec_2_w, dec_3_w, x_bar_w,
         gnn_1_w, gnn_2_w, gnn_3_w, gnn_4_w, gnn_5_w), cluster_layer)

    p_bf = {
        "enc_1": ew1, "enc_2": ew2, "enc_3": ew3, "z": zw,
        "dec_1": dw1, "dec_2": dw2, "dec_3": dw3, "x_bar": xw,
    }
    biases = {
        "enc_1": enc_1_b.reshape(1, -1), "enc_2": enc_2_b.reshape(1, -1),
        "enc_3": enc_3_b.reshape(1, -1), "z": z_b.reshape(1, -1),
        "dec_1": dec_1_b.reshape(1, -1), "dec_2": dec_2_b.reshape(1, -1),
        "dec_3": dec_3_b.reshape(1, -1), "x_bar": x_bar_b.reshape(1, -1),
    }

    x_bar, z, h1, h2, t4, t5, sup1, q = _ae_forward(
        x, p_bf, biases, gw1, gw4, gw5, c_t, sigma=sigma, v=v)

    sup2, a01, c = _gnn_first(adj, sup1, h1, gw2, sigma=sigma)
    feat2 = _gnn_chain(a01, c, sup2, h2, (), sup2.shape[1],
                       sigma=sigma, mode="feat")
    sup4 = _gnn_chain(a01, c, feat2, t4, (gw3, gw4),
                      gw4.shape[1], sigma=sigma, mode="expand")
    sup5 = _gnn_chain(a01, c, sup4, t5, (gw5,),
                      gw5.shape[1], sigma=sigma, mode="premixed")

    predict = _gnn_final(a01, c, sup5)

    return x_bar, q, predict, z


# speedup vs baseline: 1.0253x; 1.0253x over previous
import functools

import jax
import jax.numpy as jnp
from jax.experimental import pallas as pl
from jax.experimental.pallas import tpu as pltpu


_VMEM_LIMIT = 64 * 1024 * 1024


def _prep_kernel(*refs):
    k = len(refs) // 2
    in_refs, out_refs = refs[:k], refs[k:]
    for i_ref, o_ref in zip(in_refs[:-1], out_refs[:-1]):
        o_ref[...] = i_ref[...].astype(jnp.bfloat16)
    out_refs[-1][...] = jnp.transpose(in_refs[-1][...])


def _prep(weights, cluster_layer):
    ins = list(weights) + [cluster_layer]
    out_shape = tuple(jax.ShapeDtypeStruct(w.shape, jnp.bfloat16)
                      for w in weights)
    out_shape += (jax.ShapeDtypeStruct(
        (cluster_layer.shape[1], cluster_layer.shape[0]), jnp.float32),)
    return pl.pallas_call(
        _prep_kernel,
        out_shape=out_shape,
        compiler_params=pltpu.CompilerParams(
            vmem_limit_bytes=_VMEM_LIMIT,
        ),
    )(*ins)


def _row_tile(n, pref=1024):
    for t in (pref, 512, 256, 128, 64, 32, 16, 8):
        if t <= n and n % t == 0:
            return t
    return n


def _ae_kernel(x_ref,
               ew1_ref, eb1_ref, ew2_ref, eb2_ref, ew3_ref, eb3_ref,
               zw_ref, zb_ref,
               dw1_ref, db1_ref, dw2_ref, db2_ref, dw3_ref, db3_ref,
               xw_ref, xb_ref,
               gw1_ref, gw4_ref, gw5_ref, ct_ref,
               xbar_ref, z_ref, h1_ref, h2_ref, t4_ref, t5_ref,
               sup1_ref, q_ref,
               *, sigma, v):
    x = x_ref[...].astype(jnp.bfloat16)

    def lin(a_bf, w_ref, b_ref, relu):
        y = jnp.dot(a_bf, w_ref[...], preferred_element_type=jnp.float32)
        y = y + b_ref[...]
        return jnp.maximum(y, 0.0) if relu else y

    h1 = lin(x, ew1_ref, eb1_ref, True)
    h1b = h1.astype(jnp.bfloat16)
    h2 = lin(h1b, ew2_ref, eb2_ref, True)
    h2b = h2.astype(jnp.bfloat16)
    h3 = lin(h2b, ew3_ref, eb3_ref, True)
    h3b = h3.astype(jnp.bfloat16)
    z = lin(h3b, zw_ref, zb_ref, False)
    zb = z.astype(jnp.bfloat16)
    d1 = lin(zb, dw1_ref, db1_ref, True).astype(jnp.bfloat16)
    d2 = lin(d1, dw2_ref, db2_ref, True).astype(jnp.bfloat16)
    d3 = lin(d2, dw3_ref, db3_ref, True).astype(jnp.bfloat16)
    x_bar = lin(d3, xw_ref, xb_ref, False)

    xbar_ref[...] = x_bar
    z_ref[...] = z
    h1_ref[...] = h1b
    h2_ref[...] = h2b
    sup1_ref[...] = jnp.dot(x, gw1_ref[...],
                            preferred_element_type=jnp.float32).astype(jnp.bfloat16)
    t4_ref[...] = (sigma * jnp.dot(h3b, gw4_ref[...],
                                   preferred_element_type=jnp.float32)
                   ).astype(jnp.bfloat16)
    t5_ref[...] = (sigma * jnp.dot(zb, gw5_ref[...],
                                   preferred_element_type=jnp.float32)
                   ).astype(jnp.bfloat16)

    ct = ct_ref[...]
    zz = jnp.sum(z * z, axis=1, keepdims=True)
    cc = jnp.sum(ct * ct, axis=0, keepdims=True)
    d = zz + cc - 2.0 * jnp.dot(z, ct, preferred_element_type=jnp.float32)
    d = jnp.maximum(d, 0.0)
    q = 1.0 / (1.0 + d / v)
    expo = (v + 1.0) / 2.0
    if expo != 1.0:
        q = q ** expo
    q_ref[...] = q / jnp.sum(q, axis=1, keepdims=True)


def _ae_forward(x, p_bf, biases, gw1_bf, gw4_bf, gw5_bf, c_t, *, sigma, v):
    n, n_input = x.shape
    tm = _row_tile(n)
    grid = (n // tm,)

    w_names = ("enc_1", "enc_2", "enc_3", "z", "dec_1", "dec_2", "dec_3", "x_bar")
    args = [x]
    in_specs = [pl.BlockSpec((tm, n_input), lambda i: (i, 0))]
    for name in w_names:
        w = p_bf[name]
        b = biases[name]
        args += [w, b]
        in_specs += [pl.BlockSpec(w.shape, lambda i: (0, 0)),
                     pl.BlockSpec(b.shape, lambda i: (0, 0))]
    args += [gw1_bf, gw4_bf, gw5_bf, c_t]
    in_specs += [pl.BlockSpec(gw1_bf.shape, lambda i: (0, 0)),
                 pl.BlockSpec(gw4_bf.shape, lambda i: (0, 0)),
                 pl.BlockSpec(gw5_bf.shape, lambda i: (0, 0)),
                 pl.BlockSpec(c_t.shape, lambda i: (0, 0))]

    d_h1 = p_bf["enc_1"].shape[1]
    d_h2 = p_bf["enc_2"].shape[1]
    d_z = p_bf["z"].shape[1]
    d_t4 = gw4_bf.shape[1]
    d_t5 = gw5_bf.shape[1]
    d_s1 = gw1_bf.shape[1]
    n_k = c_t.shape[1]

    out_shape = (jax.ShapeDtypeStruct((n, n_input), jnp.float32),
                 jax.ShapeDtypeStruct((n, d_z), jnp.float32),
                 jax.ShapeDtypeStruct((n, d_h1), jnp.bfloat16),
                 jax.ShapeDtypeStruct((n, d_h2), jnp.bfloat16),
                 jax.ShapeDtypeStruct((n, d_t4), jnp.bfloat16),
                 jax.ShapeDtypeStruct((n, d_t5), jnp.bfloat16),
                 jax.ShapeDtypeStruct((n, d_s1), jnp.bfloat16),
                 jax.ShapeDtypeStruct((n, n_k), jnp.float32))
    out_specs = tuple(pl.BlockSpec((tm, s.shape[1]), lambda i: (i, 0))
                      for s in out_shape)

    return pl.pallas_call(
        functools.partial(_ae_kernel, sigma=sigma, v=v),
        out_shape=out_shape,
        grid_spec=pltpu.PrefetchScalarGridSpec(
            num_scalar_prefetch=0,
            grid=grid,
            in_specs=in_specs,
            out_specs=out_specs,
        ),
        compiler_params=pltpu.CompilerParams(
            dimension_semantics=("parallel",),
            vmem_limit_bytes=_VMEM_LIMIT,
        ),
    )(*args)


def _gnn_first_kernel(adj_ref, sup_ref, tra_ref, w_ref,
                      out_ref, a01_ref, c_ref, *, sigma):
    a = adj_ref[...]
    c = jnp.max(a, axis=1, keepdims=True)
    acc = jnp.dot(a.astype(jnp.bfloat16), sup_ref[...],
                  preferred_element_type=jnp.float32)
    h = jnp.maximum(acc, 0.0)
    feat = (1.0 - sigma) * h + sigma * tra_ref[...].astype(jnp.float32)
    out_ref[...] = jnp.dot(feat.astype(jnp.bfloat16), w_ref[...],
                           preferred_element_type=jnp.float32).astype(jnp.bfloat16)
    a01_ref[...] = (a > 0.0).astype(jnp.int8)
    c_ref[...] = jnp.broadcast_to(c, (c.shape[0], 8))


def _gnn_first(adj, support, tra, w_next_bf, *, sigma):
    n = adj.shape[0]
    f_sup = support.shape[1]
    f_tra = tra.shape[1]
    f_next = w_next_bf.shape[1]
    tm = _row_tile(n, pref=512)
    grid = (n // tm,)
    return pl.pallas_call(
        functools.partial(_gnn_first_kernel, sigma=sigma),
        out_shape=(jax.ShapeDtypeStruct((n, f_next), jnp.bfloat16),
                   jax.ShapeDtypeStruct((n, n), jnp.int8),
                   jax.ShapeDtypeStruct((n, 8), jnp.float32)),
        grid_spec=pltpu.PrefetchScalarGridSpec(
            num_scalar_prefetch=0,
            grid=grid,
            in_specs=[
                pl.BlockSpec((tm, n), lambda i: (i, 0)),
                pl.BlockSpec((n, f_sup), lambda i: (0, 0)),
                pl.BlockSpec((tm, f_tra), lambda i: (i, 0)),
                pl.BlockSpec((f_sup, f_next), lambda i: (0, 0)),
            ],
            out_specs=(pl.BlockSpec((tm, f_next), lambda i: (i, 0)),
                       pl.BlockSpec((tm, n), lambda i: (i, 0)),
                       pl.BlockSpec((tm, 8), lambda i: (i, 0))),
        ),
        compiler_params=pltpu.CompilerParams(
            dimension_semantics=("parallel",),
            vmem_limit_bytes=_VMEM_LIMIT,
        ),
    )(adj, support, tra, w_next_bf)


def _gnn_kernel(a01_ref, c_ref, sup_ref, tra_ref, *args, sigma, mode):
    out_ref = args[-1]
    a01 = a01_ref[...].astype(jnp.bfloat16)
    c = c_ref[:, 0:1]
    acc = jnp.dot(a01, sup_ref[...],
                  preferred_element_type=jnp.float32) * c
    if mode == "feat":
        h = jnp.maximum(acc, 0.0)
        out = (1.0 - sigma) * h + sigma * tra_ref[...].astype(jnp.float32)
    elif mode == "expand":
        win_ref, wout_ref = args[0], args[1]
        h = jnp.maximum(jnp.dot(acc.astype(jnp.bfloat16), win_ref[...],
                                preferred_element_type=jnp.float32), 0.0)
        hw = jnp.dot(h.astype(jnp.bfloat16), wout_ref[...],
                     preferred_element_type=jnp.float32)
        out = (1.0 - sigma) * hw + tra_ref[...].astype(jnp.float32)
    else:
        w_ref = args[0]
        h = jnp.maximum(acc, 0.0)
        hw = jnp.dot(h.astype(jnp.bfloat16), w_ref[...],
                     preferred_element_type=jnp.float32)
        out = (1.0 - sigma) * hw + tra_ref[...].astype(jnp.float32)
    out_ref[...] = out.astype(jnp.bfloat16)


def _gnn_chain(a01, c, support, tra, weights, f_next, *, sigma, mode):
    n = a01.shape[0]
    f_sup = support.shape[1]
    f_tra = tra.shape[1]
    tm = _row_tile(n)
    grid = (n // tm,)
    in_specs = [
        pl.BlockSpec((tm, n), lambda i: (i, 0)),
        pl.BlockSpec((tm, 8), lambda i: (i, 0)),
        pl.BlockSpec((n, f_sup), lambda i: (0, 0)),
        pl.BlockSpec((tm, f_tra), lambda i: (i, 0)),
    ]
    for w in weights:
        in_specs.append(pl.BlockSpec(w.shape, lambda i: (0, 0)))
    return pl.pallas_call(
        functools.partial(_gnn_kernel, sigma=sigma, mode=mode),
        out_shape=jax.ShapeDtypeStruct((n, f_next), jnp.bfloat16),
        grid_spec=pltpu.PrefetchScalarGridSpec(
            num_scalar_prefetch=0,
            grid=grid,
            in_specs=in_specs,
            out_specs=pl.BlockSpec((tm, f_next), lambda i: (i, 0)),
        ),
        compiler_params=pltpu.CompilerParams(
            dimension_semantics=("parallel",),
            vmem_limit_bytes=_VMEM_LIMIT,
        ),
    )(a01, c, support, tra, *weights)


def _final_kernel(a01_ref, c_ref, sup_ref, pred_ref):
    a01 = a01_ref[...].astype(jnp.bfloat16)
    c = c_ref[:, 0:1]
    h = jnp.dot(a01, sup_ref[...],
                preferred_element_type=jnp.float32) * c
    m = jnp.max(h, axis=1, keepdims=True)
    e = jnp.exp(h - m)
    pred_ref[...] = e / jnp.sum(e, axis=1, keepdims=True)


def _gnn_final(a01, c, support):
    n = a01.shape[0]
    f_sup = support.shape[1]
    tm = _row_tile(n)
    grid = (n // tm,)
    return pl.pallas_call(
        _final_kernel,
        out_shape=jax.ShapeDtypeStruct((n, f_sup), jnp.float32),
        grid_spec=pltpu.PrefetchScalarGridSpec(
            num_scalar_prefetch=0,
            grid=grid,
            in_specs=[
                pl.BlockSpec((tm, n), lambda i: (i, 0)),
                pl.BlockSpec((tm, 8), lambda i: (i, 0)),
                pl.BlockSpec((n, f_sup), lambda i: (0, 0)),
            ],
            out_specs=pl.BlockSpec((tm, f_sup), lambda i: (i, 0)),
        ),
        compiler_params=pltpu.CompilerParams(
            dimension_semantics=("parallel",),
            vmem_limit_bytes=_VMEM_LIMIT,
        ),
    )(a01, c, support)


def kernel(x, adj, enc_1_w, enc_1_b, enc_2_w, enc_2_b, enc_3_w, enc_3_b,
           z_w, z_b, dec_1_w, dec_1_b, dec_2_w, dec_2_b, dec_3_w, dec_3_b,
           x_bar_w, x_bar_b,
           gnn_1_w, gnn_2_w, gnn_3_w, gnn_4_w, gnn_5_w,
           cluster_layer):
    sigma, v = 0.5, 1.0

    (ew1, ew2, ew3, zw, dw1, dw2, dw3, xw,
     gw1, gw2, gw3, gw4, gw5, c_t) = _prep(
        (enc_1_w, enc_2_w, enc_3_w, z_w, dec_1_w, dec_2_w, dec_3_w, x_bar_w,
         gnn_1_w, gnn_2_w, gnn_3_w, gnn_4_w, gnn_5_w), cluster_layer)

    p_bf = {
        "enc_1": ew1, "enc_2": ew2, "enc_3": ew3, "z": zw,
        "dec_1": dw1, "dec_2": dw2, "dec_3": dw3, "x_bar": xw,
    }
    biases = {
        "enc_1": enc_1_b.reshape(1, -1), "enc_2": enc_2_b.reshape(1, -1),
        "enc_3": enc_3_b.reshape(1, -1), "z": z_b.reshape(1, -1),
        "dec_1": dec_1_b.reshape(1, -1), "dec_2": dec_2_b.reshape(1, -1),
        "dec_3": dec_3_b.reshape(1, -1), "x_bar": x_bar_b.reshape(1, -1),
    }

    x_bar, z, h1, h2, t4, t5, sup1, q = _ae_forward(
        x, p_bf, biases, gw1, gw4, gw5, c_t, sigma=sigma, v=v)

    sup2, a01, c = _gnn_first(adj, sup1, h1, gw2, sigma=sigma)
    feat2 = _gnn_chain(a01, c, sup2, h2, (), sup2.shape[1],
                       sigma=sigma, mode="feat")
    sup4 = _gnn_chain(a01, c, feat2, t4, (gw3, gw4),
                      gw4.shape[1], sigma=sigma, mode="expand")
    sup5 = _gnn_chain(a01, c, sup4, t5, (gw5,),
                      gw5.shape[1], sigma=sigma, mode="premixed")

    predict = _gnn_final(a01, c, sup5)

    return x_bar, q, predict, z
